# baseline (speedup 1.0000x reference)
import functools

import jax
import jax.numpy as jnp
from jax import lax
from jax.experimental import pallas as pl
from jax.experimental.pallas import tpu as pltpu
from jax.experimental.pallas import tpu_sc as plsc

_B = 128
_V = 100000
_C = 0.1
_NC = 2
_NS = 16
_NW = _NC * _NS
_RPW = _B // _NW
_WORDS = _RPW * _V
_CH = 20000
_CHPR = _V // _CH
_NCH = _WORDS // _CH
_NBUF = 4
_UNROLL = 8
_FULL = (_CH // (16 * _UNROLL)) * (16 * _UNROLL)
_NIT = _FULL // (16 * _UNROLL)
_NTAIL = (_CH - _FULL) // 16


def _sc_body(tgt_hbm, predf_hbm, out_hbm, tgt_v, idx_v, corr_v, bufs_v,
             tot_v, csem, *dsems):
    cid = lax.axis_index("c")
    sid = lax.axis_index("s")
    wid = sid * _NC + cid
    base = wid * _WORDS

    pltpu.sync_copy(tgt_hbm, tgt_v)
    lanes = lax.iota(jnp.int32, 16)
    rloc = jnp.minimum(lanes, _RPW - 1)
    rows = wid * _RPW + rloc
    t16 = plsc.load_gather(tgt_v, [rows])
    idx_v[...] = rows * _V + t16
    pltpu.async_copy(predf_hbm.at[idx_v], corr_v, csem).wait()

    def chunk_src(j):
        return predf_hbm.at[pl.ds(base + j * _CH, _CH)]

    handles = {}
    for j in range(_NBUF):
        handles[j] = pltpu.async_copy(
            chunk_src(j), bufs_v.at[pl.ds(j * _CH, _CH)], dsems[j])

    accs = tuple(jnp.zeros((16,), jnp.float32) for _ in range(_UNROLL))
    for j in range(_NCH):
        handles[j].wait()
        boff = (j % _NBUF) * _CH
        shift16 = plsc.load_gather(
            corr_v, [jnp.full((16,), j // _CHPR, jnp.int32)]) - _C

        def body(i, a, boff=boff, shift16=shift16):
            off = boff + i * (16 * _UNROLL)
            return tuple(
                a[k] + jnp.maximum(
                    bufs_v[pl.ds(off + k * 16, 16)] - shift16, 0.0)
                for k in range(_UNROLL))

        accs = lax.fori_loop(0, _NIT, body, accs)
        for k in range(_NTAIL):
            x = bufs_v[pl.ds(boff + _FULL + k * 16, 16)]
            accs = tuple(
                (a + jnp.maximum(x - shift16, 0.0)) if kk == k else a
                for kk, a in enumerate(accs))
        nxt = j + _NBUF
        if nxt < _NCH:
            handles[nxt] = pltpu.async_copy(
                chunk_src(nxt), bufs_v.at[pl.ds(boff, _CH)], dsems[j % _NBUF])

    tot = accs[0]
    for k in range(1, _UNROLL):
        tot = tot + accs[k]
    tot_v[...] = tot
    pltpu.sync_copy(tot_v, out_hbm.at[wid])


_sc_loss = functools.partial(
    pl.kernel,
    out_type=jax.ShapeDtypeStruct((_NW, 16), jnp.float32),
    mesh=plsc.VectorSubcoreMesh(core_axis_name="c", subcore_axis_name="s",
                                num_cores=_NC, num_subcores=_NS),
    scratch_types=[
        pltpu.VMEM((_B,), jnp.int32),
        pltpu.VMEM((16,), jnp.int32),
        pltpu.VMEM((16,), jnp.float32),
        pltpu.VMEM((_NBUF * _CH,), jnp.float32),
        pltpu.VMEM((16,), jnp.float32),
        pltpu.SemaphoreType.DMA,
    ] + [pltpu.SemaphoreType.DMA] * _NBUF,
    compiler_params=pltpu.CompilerParams(needs_layout_passes=False),
)(_sc_body)


def kernel(target, prediction):
    partials = _sc_loss(target.astype(jnp.int32), prediction.reshape(-1))
    return ((jnp.sum(partials) - _B * _C) / _B).reshape((1,))


# speedup vs baseline: 2.2655x; 2.2655x over previous
import jax
import jax.numpy as jnp
from jax.experimental import pallas as pl
from jax.experimental.pallas import tpu as pltpu

_B = 128
_V = 100000
_RB = 16
_NSTREAM = 2
_C = 0.1


def _loss_kernel(targets_ref, xa_ref, xb_ref, out_ref):
    b = pl.program_id(0)
    lane_ids = jax.lax.broadcasted_iota(jnp.int32, (1, 128), 1)

    def block_sum(x_ref, base):
        cs = []
        for r in range(_RB):
            t = targets_ref[base + r]
            chunk_start = pl.multiple_of((t // 128) * 128, 128)
            chunk = x_ref[r, pl.ds(chunk_start, 128)].reshape(1, 128)
            lane = t % 128
            cs.append(jnp.sum(jnp.where(lane_ids == lane, chunk, 0.0)))
        correct = jnp.stack(cs).reshape(_RB, 1)
        return jnp.sum(jnp.maximum(x_ref[...] - (correct - _C), 0.0))

    s = (block_sum(xa_ref, b * _NSTREAM * _RB)
         + block_sum(xb_ref, b * _NSTREAM * _RB + _RB))

    @pl.when(b == 0)
    def _init():
        out_ref[...] = jnp.zeros_like(out_ref)

    out_ref[...] += s

    @pl.when(b == (_B // (_NSTREAM * _RB)) - 1)
    def _finish():
        out_ref[...] = (out_ref[...] - _B * _C) / _B


def kernel(target, prediction):
    target = target.astype(jnp.int32)
    out = pl.pallas_call(
        _loss_kernel,
        grid_spec=pltpu.PrefetchScalarGridSpec(
            num_scalar_prefetch=1,
            grid=(_B // (_NSTREAM * _RB),),
            in_specs=[
                pl.BlockSpec((_RB, _V), lambda i, t: (_NSTREAM * i, 0)),
                pl.BlockSpec((_RB, _V), lambda i, t: (_NSTREAM * i + 1, 0)),
            ],
            out_specs=pl.BlockSpec((1, 1), lambda i, t: (0, 0)),
        ),
        out_shape=jax.ShapeDtypeStruct((1, 1), jnp.float32),
        compiler_params=pltpu.CompilerParams(
            dimension_semantics=("arbitrary",),
        ),
    )(target, prediction, prediction)
    return out.reshape((1,))
